# baseline (speedup 1.0000x reference)
import functools

import jax
import jax.numpy as jnp
from jax import lax
from jax.experimental import pallas as pl
from jax.experimental.pallas import tpu as pltpu
from jax.experimental.pallas import tpu_sc as plsc

NUM_ROWS = 1000000
D = 32
NC, NS = 2, 16
NW = NC * NS
B_TOTAL = 16384 * 26
BPW = B_TOTAL // NW
SR = NUM_ROWS // 4
SB = 512
NSB = (NUM_ROWS - 64) // SB
TAIL0 = NUM_ROWS - 64
NU = 104

_params = pltpu.CompilerParams(use_tc_tiling_on_sc=True, needs_layout_passes=False)
_mesh = plsc.VectorSubcoreMesh(core_axis_name="c", subcore_axis_name="s")


def _wid():
    return lax.axis_index("s") * NC + lax.axis_index("c")


@functools.partial(
    pl.kernel,
    out_type=jax.ShapeDtypeStruct((SR, 128), jnp.float32),
    mesh=_mesh,
    scratch_types=[
        pltpu.VMEM((D, SB), jnp.float32),
        pltpu.VMEM((D, SB), jnp.float32),
        pltpu.VMEM((128, 128), jnp.float32),
        pltpu.VMEM((128, 128), jnp.float32),
        pltpu.SemaphoreType.DMA,
        pltpu.SemaphoreType.DMA,
        pltpu.SemaphoreType.DMA,
        pltpu.SemaphoreType.DMA,
    ],
    compiler_params=_params,
)
def _relayout(tt, scratch, tb0, tb1, sb0, sb1, tsem0, tsem1, osem0, osem1):
    w = _wid()
    s0 = 61 * w + jnp.minimum(w, 1)
    n = jnp.where(w == 0, 62, 61)
    i16 = lax.iota(jnp.int32, 16)

    def _fire_in(s, b):
        src = tt.at[:, pl.ds(pl.multiple_of(s * SB, SB), SB)]

        @pl.when(b == 0)
        def _():
            pltpu.async_copy(src, tb0, tsem0)

        @pl.when(b == 1)
        def _():
            pltpu.async_copy(src, tb1, tsem1)

    def _wait_in(b):
        @pl.when(b == 0)
        def _():
            pltpu.make_async_copy(tt.at[:, pl.ds(0, SB)], tb0, tsem0).wait()

        @pl.when(b == 1)
        def _():
            pltpu.make_async_copy(tt.at[:, pl.ds(0, SB)], tb1, tsem1).wait()

    def _wait_out(b):
        @pl.when(b == 0)
        def _():
            pltpu.make_async_copy(scratch.at[pl.ds(0, 128), :], sb0, osem0).wait()

        @pl.when(b == 1)
        def _():
            pltpu.make_async_copy(scratch.at[pl.ds(0, 128), :], sb1, osem1).wait()

    def _fire_out(s, b):
        dst = scratch.at[pl.ds(pl.multiple_of(s * 128, 128), 128), :]

        @pl.when(b == 0)
        def _():
            pltpu.async_copy(sb0, dst, osem0)

        @pl.when(b == 1)
        def _():
            pltpu.async_copy(sb1, dst, osem1)

    _fire_in(s0, 0)
    r0 = lax.shift_right_logical(i16, 2)
    c0 = lax.bitwise_and(i16, 3) * D

    def _transpose(tb, sb):
        for m in range(32):
            rowv = r0 + 4 * m
            for d in range(D):
                plsc.store_scatter(sb, [rowv, c0 + d], tb[d, pl.ds(16 * m, 16)])

    @pl.loop(0, n)
    def _blk(k):
        b = lax.rem(k, 2)
        s = s0 + k
        _wait_in(b)

        @pl.when(k + 1 < n)
        def _():
            _fire_in(s + 1, 1 - b)

        @pl.when(k >= 2)
        def _():
            _wait_out(b)

        @pl.when(b == 0)
        def _():
            _transpose(tb0, sb0)

        @pl.when(b == 1)
        def _():
            _transpose(tb1, sb1)

        _fire_out(s, b)

    pltpu.make_async_copy(scratch.at[pl.ds(0, 128), :], sb0, osem0).wait()
    pltpu.make_async_copy(scratch.at[pl.ds(0, 128), :], sb1, osem1).wait()


@functools.partial(
    pl.kernel,
    out_type=jax.ShapeDtypeStruct((26, D, 16384), jnp.float32),
    mesh=_mesh,
    scratch_types=[
        pltpu.VMEM((104, 128), jnp.int32),
        pltpu.VMEM((64, 128), jnp.float32),
        pltpu.VMEM((2, 128), jnp.int32),
        pltpu.VMEM((2, 128), jnp.int32),
        pltpu.VMEM((128, 128), jnp.float32),
        pltpu.VMEM((128, 128), jnp.float32),
        pltpu.VMEM((D, 128), jnp.float32),
        pltpu.VMEM((D, 128), jnp.float32),
        pltpu.SemaphoreType.DMA,
        pltpu.SemaphoreType.DMA,
        pltpu.SemaphoreType.DMA,
        pltpu.SemaphoreType.DMA,
    ],
    compiler_params=_params,
)
def _gather(xf, scratch, tailp, outp, xbuf, tailb, gidx, ibuf, dst0, dst1,
            obuf0, obuf1, gsem0, gsem1, osem0, osem1):
    w = _wid()
    i16 = lax.iota(jnp.int32, 16)
    i26 = i16 * 26

    pltpu.sync_copy(xf.at[pl.ds(pl.multiple_of(w * 104, 8), 104), :], xbuf)
    pltpu.sync_copy(tailp, tailb)

    def _prep(u, b):
        cl = u // 26
        j = u - cl * 26
        for g in range(8):
            base = (cl * 128 + g * 16) * 26 + j
            pvec = i26 + base
            iv = plsc.load_gather(xbuf, [lax.shift_right_logical(pvec, 7),
                                         lax.bitwise_and(pvec, 127)])
            gidx[b, pl.ds(16 * g, 16)] = lax.shift_right_logical(iv, 2)
            ibuf[b, pl.ds(16 * g, 16)] = iv

    def _fire_gather(b):
        @pl.when(b == 0)
        def _():
            pltpu.async_copy(scratch.at[gidx.at[0]], dst0, gsem0)

        @pl.when(b == 1)
        def _():
            pltpu.async_copy(scratch.at[gidx.at[1]], dst1, gsem1)

    def _wait_gather(b):
        @pl.when(b == 0)
        def _():
            pltpu.make_async_copy(scratch.at[pl.ds(0, 128), :], dst0, gsem0).wait()

        @pl.when(b == 1)
        def _():
            pltpu.make_async_copy(scratch.at[pl.ds(0, 128), :], dst1, gsem1).wait()

    def _wait_out(b):
        @pl.when(b == 0)
        def _():
            pltpu.make_async_copy(outp.at[0, :, pl.ds(0, 128)], obuf0, osem0).wait()

        @pl.when(b == 1)
        def _():
            pltpu.make_async_copy(outp.at[0, :, pl.ds(0, 128)], obuf1, osem1).wait()

    def _fire_out(u, b):
        cl = u // 26
        j = u - cl * 26
        cb = 4 * w + cl
        dstref = outp.at[j, :, pl.ds(pl.multiple_of(cb * 128, 128), 128)]

        @pl.when(b == 0)
        def _():
            pltpu.async_copy(obuf0, dstref, osem0)

        @pl.when(b == 1)
        def _():
            pltpu.async_copy(obuf1, dstref, osem1)

    def _extract_into(b, dstb, obufb):
        tmax = jnp.zeros((16,), jnp.int32)
        for g in range(8):
            ivg = ibuf[b, pl.ds(16 * g, 16)]
            tmax = jnp.maximum(tmax, jnp.where(ivg >= TAIL0, 1, 0))
            remg = lax.bitwise_and(ivg, 3) * D
            ccv = i16 + 16 * g
            for dd in range(D):
                v = plsc.load_gather(dstb, [ccv, remg + dd])
                obufb[dd, pl.ds(16 * g, 16)] = v

        @pl.when(lax.reduce_max(tmax, (0,)) > 0)
        def _():
            for g in range(8):
                ivg = ibuf[b, pl.ds(16 * g, 16)]
                mv = ivg >= TAIL0
                tg = jnp.clip(ivg - TAIL0, 0, 63)
                for dd in range(D):
                    tv = plsc.load_gather(tailb, [tg, jnp.full((16,), dd, jnp.int32)])
                    cur = obufb[dd, pl.ds(16 * g, 16)]
                    obufb[dd, pl.ds(16 * g, 16)] = jnp.where(mv, tv, cur)

    def _extract(b):
        @pl.when(b == 0)
        def _():
            _extract_into(b, dst0, obuf0)

        @pl.when(b == 1)
        def _():
            _extract_into(b, dst1, obuf1)

    _prep(0, 0)
    _fire_gather(0)

    @pl.loop(0, NU)
    def _unit(u):
        b = lax.rem(u, 2)
        _wait_gather(b)

        @pl.when(u + 1 < NU)
        def _():
            _prep(u + 1, 1 - b)
            _fire_gather(1 - b)

        @pl.when(u >= 2)
        def _():
            _wait_out(b)

        _extract(b)
        _fire_out(u, b)

    pltpu.make_async_copy(outp.at[0, :, pl.ds(0, 128)], obuf0, osem0).wait()
    pltpu.make_async_copy(outp.at[0, :, pl.ds(0, 128)], obuf1, osem1).wait()


def kernel(x, table):
    xf = x.reshape(B_TOTAL // 128, 128).astype(jnp.int32)
    tailp = jnp.pad(
        lax.slice(table, (TAIL0, 0), (NUM_ROWS, D)), ((0, 0), (0, 128 - D))
    )
    scratch = _relayout(table.T)
    outp = _gather(xf, scratch, tailp)
    return jnp.transpose(outp, (2, 0, 1))


# speedup vs baseline: 1.2651x; 1.2651x over previous
import functools

import jax
import jax.numpy as jnp
from jax import lax
from jax.experimental import pallas as pl
from jax.experimental.pallas import tpu as pltpu
from jax.experimental.pallas import tpu_sc as plsc

NUM_ROWS = 1000000
D = 32

NC, NS = 2, 16
NW = NC * NS
G = 128
B_TOTAL = 16384 * 26
B_PER_W = B_TOTAL // NW
C = 1664
NCHUNK = B_PER_W // C
GPC = C // G
IDX_ROWS = B_PER_W // G


@functools.partial(
    pl.kernel,
    out_type=jax.ShapeDtypeStruct((B_TOTAL, D), jnp.float32),
    mesh=plsc.VectorSubcoreMesh(core_axis_name="c", subcore_axis_name="s"),
    scratch_types=[
        pltpu.VMEM((IDX_ROWS, G), jnp.int32),
        pltpu.VMEM((2, C, D), jnp.float32),
        pltpu.SemaphoreType.DMA,
        pltpu.SemaphoreType.DMA,
        pltpu.SemaphoreType.DMA,
    ],
    compiler_params=pltpu.CompilerParams(use_tc_tiling_on_sc=False),
)
def _gather_kernel(idx_hbm, table_hbm, out_hbm, idx_v, rows_v, gsem,
                   osem0, osem1):
    wid = lax.axis_index("s") * NC + lax.axis_index("c")
    row_base = wid * B_PER_W

    pltpu.sync_copy(
        idx_hbm.at[pl.ds(pl.multiple_of(row_base // G, IDX_ROWS), IDX_ROWS)],
        idx_v,
    )

    @pl.loop(0, NCHUNK)
    def _chunk(ci):
        b = lax.rem(ci, 2)
        off = pl.multiple_of(row_base + ci * C, C)

        @pl.when(ci >= 2)
        def _():
            @pl.when(b == 0)
            def _():
                pltpu.make_async_copy(
                    out_hbm.at[pl.ds(0, C)], rows_v.at[0], osem0
                ).wait()

            @pl.when(b == 1)
            def _():
                pltpu.make_async_copy(
                    out_hbm.at[pl.ds(0, C)], rows_v.at[1], osem1
                ).wait()

        copies = []
        for j in range(GPC):
            copies.append(
                pltpu.async_copy(
                    table_hbm.at[idx_v.at[ci * GPC + j]],
                    rows_v.at[b, pl.ds(j * G, G)],
                    gsem,
                )
            )
        for cp in copies:
            cp.wait()

        @pl.when(b == 0)
        def _():
            pltpu.async_copy(rows_v.at[0], out_hbm.at[pl.ds(off, C)], osem0)

        @pl.when(b == 1)
        def _():
            pltpu.async_copy(rows_v.at[1], out_hbm.at[pl.ds(off, C)], osem1)

    pltpu.make_async_copy(out_hbm.at[pl.ds(0, C)], rows_v.at[0], osem0).wait()
    pltpu.make_async_copy(out_hbm.at[pl.ds(0, C)], rows_v.at[1], osem1).wait()


def kernel(x, table):
    idx = x.reshape(B_TOTAL // G, G).astype(jnp.int32)
    out = _gather_kernel(idx, table)
    return out.reshape(x.shape + (D,))
